# baseline (speedup 1.0000x reference)
import functools

import jax
import jax.numpy as jnp
from jax import lax
from jax.experimental import pallas as pl
from jax.experimental.pallas import tpu as pltpu
from jax.experimental.pallas import tpu_sc as plsc

B = 16384
L = 50
D = 32
V = 1000000
N = B * L

_info = plsc.get_sparse_core_info()
NC, NS = _info.num_cores, _info.num_subcores
NW = NC * NS
PER_W = N // NW
CH = 128
N_CH = PER_W // CH
SUP = 2
N_SUP = N_CH // SUP
N_PAIR = N_SUP // 2
PV = V // 4
FULL_T = (V // CH)
TAIL = V - FULL_T * CH

_mesh = plsc.VectorSubcoreMesh(core_axis_name="c", subcore_axis_name="s")
_params = pltpu.CompilerParams(needs_layout_passes=False)


@functools.partial(
    pl.kernel,
    mesh=_mesh,
    compiler_params=_params,
    out_type=jax.ShapeDtypeStruct((PV, CH), jnp.float32),
    scratch_types=[
        pltpu.VMEM((2, D, CH), jnp.float32),
        pltpu.VMEM((2, D, CH), jnp.float32),
        pltpu.SemaphoreType.DMA,
        pltpu.SemaphoreType.DMA,
        pltpu.SemaphoreType.DMA,
        pltpu.SemaphoreType.DMA,
    ],
)
def _repack(table_t_hbm, tail_hbm, packed_hbm, s_buf, d_buf,
            rsem0, rsem1, wsem0, wsem1):
    wid = lax.axis_index("s") * NC + lax.axis_index("c")
    rsem = (rsem0, rsem1)
    wsem = (wsem0, wsem1)

    iota = lax.iota(jnp.int32, 16)
    d_vecs = []
    q_vecs = []
    for cg in range(CH // 16):
        col = cg * 16 + iota
        d_vecs.append(col % D)
        q_vecs.append(col // D)

    n_t = jnp.where(wid < FULL_T % NW, FULL_T // NW + 1, FULL_T // NW)

    def rd(t, b):
        return pltpu.make_async_copy(
            table_t_hbm.at[:, pl.ds(t * CH, CH)], s_buf.at[b], rsem[b])

    def wr(t, b):
        return pltpu.make_async_copy(
            d_buf.at[b], packed_hbm.at[pl.ds(t * D, D), :], wsem[b])

    def transpose(sbuf, dbuf):
        def one_row(r, carry):
            for cg in range(CH // 16):
                dbuf[r, pl.ds(cg * 16, 16)] = plsc.load_gather(
                    sbuf, [d_vecs[cg], q_vecs[cg] + 4 * r])
            return carry
        lax.fori_loop(0, D, one_row, 0)

    rd(wid, 0).start()

    def step(i, t, b):
        @pl.when(i + 1 < n_t)
        def _():
            rd(t + NW, 1 - b).start()
        rd(t, b).wait()

        @pl.when(i > 1)
        def _():
            wr(t - 2 * NW, b).wait()
        transpose(s_buf.at[b], d_buf.at[b])
        wr(t, b).start()

    def body(i, carry):
        t = wid + i * NW

        @pl.when(lax.rem(i, 2) == 0)
        def _():
            step(i, t, 0)

        @pl.when(lax.rem(i, 2) == 1)
        def _():
            step(i, t, 1)
        return carry

    lax.fori_loop(0, n_t, body, 0)

    wr(wid, 0).wait()
    wr(wid, 1).wait()

    @pl.when(wid == NW - 1)
    def _():
        pltpu.sync_copy(tail_hbm, s_buf)

        def one_row(r, carry):
            for cg in range(CH // 16):
                vt = q_vecs[cg] + 4 * r
                d_buf[0, r, pl.ds(cg * 16, 16)] = plsc.load_gather(
                    s_buf, [vt // D, lax.rem(vt, D), d_vecs[cg]])
            return carry
        lax.fori_loop(0, TAIL // 4, one_row, 0)
        pltpu.sync_copy(d_buf.at[0, pl.ds(0, TAIL // 4), :],
                        packed_hbm.at[pl.ds(FULL_T * D, TAIL // 4), :])


@functools.partial(
    pl.kernel,
    mesh=_mesh,
    compiler_params=_params,
    out_type=jax.ShapeDtypeStruct((L, D, B), jnp.float32),
    scratch_types=[
        pltpu.VMEM((N_CH, CH), jnp.int32),
        pltpu.VMEM((2, SUP, CH), jnp.int32),
        pltpu.VMEM((SUP * CH, CH), jnp.float32),
        pltpu.VMEM((SUP * CH, CH), jnp.float32),
        pltpu.VMEM((SUP, D, CH), jnp.float32),
        pltpu.VMEM((SUP, D, CH), jnp.float32),
        pltpu.SemaphoreType.DMA,
        pltpu.SemaphoreType.DMA,
        pltpu.SemaphoreType.DMA,
        pltpu.SemaphoreType.DMA,
    ],
)
def _lookup(idx_hbm, packed_hbm, out_hbm, idx_v, pr_v, g_x, g_y, t_x, t_y,
            gsem_x, gsem_y, ssem_x, ssem_y):
    wid = lax.axis_index("s") * NC + lax.axis_index("c")

    pltpu.sync_copy(idx_hbm.at[pl.ds(wid * N_CH, N_CH)], idx_v)

    iota = lax.iota(jnp.int32, 16)

    def fire_gathers(sg, xy, buf, sem):
        hs = []
        for c in range(SUP):
            k = sg * SUP + c
            for jg in range(CH // 16):
                v = idx_v[k, pl.ds(jg * 16, 16)]
                pr_v[xy, c, pl.ds(jg * 16, 16)] = v // 4
            hs.append(pltpu.make_async_copy(
                packed_hbm.at[pr_v.at[xy, c]],
                buf.at[pl.ds(c * CH, CH)], sem))
        for h in hs:
            h.start()

    def drain_gathers(sg, xy, buf, sem):
        for c in range(SUP):
            pltpu.make_async_copy(
                packed_hbm.at[pr_v.at[xy, c]],
                buf.at[pl.ds(c * CH, CH)], sem).wait()

    def scatters(sg, tbuf, sem):
        hs = []
        for c in range(SUP):
            k = wid * N_CH + sg * SUP + c
            row_l = k // CH
            col = (k % CH) * CH
            hs.append(pltpu.make_async_copy(
                tbuf.at[c], out_hbm.at[row_l, :, pl.ds(col, CH)], sem))
        return hs

    def transpose(sg, gbuf, tbuf):
        for c in range(SUP):
            k = sg * SUP + c

            def one_group(jg, carry, c=c, k=k):
                v = idx_v[k, pl.ds(jg * 16, 16)]
                colbase = (v % 4) * D
                rows = c * CH + jg * 16 + iota
                for d in range(D):
                    tbuf[c, d, pl.ds(jg * 16, 16)] = plsc.load_gather(
                        gbuf, [rows, colbase + d])
                return carry
            lax.fori_loop(0, CH // 16, one_group, 0)

    fire_gathers(0, 0, g_x, gsem_x)

    def body(i, carry):
        s0 = 2 * i
        s1 = 2 * i + 1

        @pl.when(i > 0)
        def _():
            for h in scatters(s1 - 2, t_y, ssem_y):
                h.wait()

        fire_gathers(s1, 1, g_y, gsem_y)
        drain_gathers(s0, 0, g_x, gsem_x)

        @pl.when(i > 0)
        def _():
            for h in scatters(s0 - 2, t_x, ssem_x):
                h.wait()

        transpose(s0, g_x, t_x)
        for h in scatters(s0, t_x, ssem_x):
            h.start()

        @pl.when(i < N_PAIR - 1)
        def _():
            fire_gathers(s0 + 2, 0, g_x, gsem_x)

        drain_gathers(s1, 1, g_y, gsem_y)
        transpose(s1, g_y, t_y)
        for h in scatters(s1, t_y, ssem_y):
            h.start()
        return carry

    lax.fori_loop(0, N_PAIR, body, 0)

    for h in scatters(N_SUP - 2, t_x, ssem_x):
        h.wait()
    for h in scatters(N_SUP - 1, t_y, ssem_y):
        h.wait()


def kernel(vocab_ids, table):
    idx = jnp.transpose(vocab_ids).reshape(N // CH, CH)
    tail = jnp.pad(table[FULL_T * CH:], ((0, 0), (0, CH - D)))
    packed = _repack(jnp.transpose(table), tail.reshape(2, D, CH))
    out_t = _lookup(idx, packed)
    return out_t.transpose(2, 0, 1)


# speedup vs baseline: 1.0660x; 1.0660x over previous
import functools

import jax
import jax.numpy as jnp
from jax import lax
from jax.experimental import pallas as pl
from jax.experimental.pallas import tpu as pltpu
from jax.experimental.pallas import tpu_sc as plsc

B = 16384
L = 50
D = 32
V = 1000000
N = B * L

_info = plsc.get_sparse_core_info()
NC, NS = _info.num_cores, _info.num_subcores
NW = NC * NS
PER_W = N // NW
CH = 128
N_CH = PER_W // CH
SUP = 1
N_SUP = N_CH // SUP
N_PAIR = N_SUP // 2
PV = V // 4
FULL_T = (V // CH)
TAIL = V - FULL_T * CH

_mesh = plsc.VectorSubcoreMesh(core_axis_name="c", subcore_axis_name="s")
_params = pltpu.CompilerParams(needs_layout_passes=False)


@functools.partial(
    pl.kernel,
    mesh=_mesh,
    compiler_params=_params,
    out_type=jax.ShapeDtypeStruct((PV, CH), jnp.float32),
    scratch_types=[
        pltpu.VMEM((2, D, CH), jnp.float32),
        pltpu.VMEM((2, D, CH), jnp.float32),
        pltpu.SemaphoreType.DMA,
        pltpu.SemaphoreType.DMA,
        pltpu.SemaphoreType.DMA,
        pltpu.SemaphoreType.DMA,
    ],
)
def _repack(table_t_hbm, tail_hbm, packed_hbm, s_buf, d_buf,
            rsem0, rsem1, wsem0, wsem1):
    wid = lax.axis_index("s") * NC + lax.axis_index("c")
    rsem = (rsem0, rsem1)
    wsem = (wsem0, wsem1)

    iota = lax.iota(jnp.int32, 16)
    r_vecs = []
    c_vecs = []
    for jg in range(CH // 16):
        j = jg * 16 + iota
        r_vecs.append(j // 4)
        c_vecs.append((j % 4) * D)
    d_vecs = []
    q_vecs = []
    for cg in range(CH // 16):
        col = cg * 16 + iota
        d_vecs.append(col % D)
        q_vecs.append(col // D)

    n_t = jnp.where(wid < FULL_T % NW, FULL_T // NW + 1, FULL_T // NW)

    def rd(t, b):
        return pltpu.make_async_copy(
            table_t_hbm.at[:, pl.ds(t * CH, CH)], s_buf.at[b], rsem[b])

    def wr(t, b):
        return pltpu.make_async_copy(
            d_buf.at[b], packed_hbm.at[pl.ds(t * D, D), :], wsem[b])

    def transpose(b):
        for d in range(D):
            for jg in range(CH // 16):
                vec = s_buf[b, d, pl.ds(jg * 16, 16)]
                plsc.store_scatter(
                    d_buf.at[b], [r_vecs[jg], c_vecs[jg] + d], vec)

    rd(wid, 0).start()

    def step(i, t, b):
        @pl.when(i + 1 < n_t)
        def _():
            rd(t + NW, 1 - b).start()
        rd(t, b).wait()

        @pl.when(i > 1)
        def _():
            wr(t - 2 * NW, b).wait()
        transpose(b)
        wr(t, b).start()

    def body(i, carry):
        t = wid + i * NW

        @pl.when(lax.rem(i, 2) == 0)
        def _():
            step(i, t, 0)

        @pl.when(lax.rem(i, 2) == 1)
        def _():
            step(i, t, 1)
        return carry

    lax.fori_loop(0, n_t, body, 0)

    wr(wid, 0).wait()
    wr(wid, 1).wait()

    @pl.when(wid == NW - 1)
    def _():
        pltpu.sync_copy(tail_hbm, s_buf)

        def one_row(r, carry):
            for cg in range(CH // 16):
                vt = q_vecs[cg] + 4 * r
                d_buf[0, r, pl.ds(cg * 16, 16)] = plsc.load_gather(
                    s_buf, [vt // D, lax.rem(vt, D), d_vecs[cg]])
            return carry
        lax.fori_loop(0, TAIL // 4, one_row, 0)
        pltpu.sync_copy(d_buf.at[0, pl.ds(0, TAIL // 4), :],
                        packed_hbm.at[pl.ds(FULL_T * D, TAIL // 4), :])


@functools.partial(
    pl.kernel,
    mesh=_mesh,
    compiler_params=_params,
    out_type=jax.ShapeDtypeStruct((L, D, B), jnp.float32),
    scratch_types=[
        pltpu.VMEM((N_CH, CH), jnp.int32),
        pltpu.VMEM((2, SUP, CH), jnp.int32),
        pltpu.VMEM((SUP * CH, CH), jnp.float32),
        pltpu.VMEM((SUP * CH, CH), jnp.float32),
        pltpu.VMEM((SUP, D, CH), jnp.float32),
        pltpu.VMEM((SUP, D, CH), jnp.float32),
        pltpu.SemaphoreType.DMA,
        pltpu.SemaphoreType.DMA,
        pltpu.SemaphoreType.DMA,
        pltpu.SemaphoreType.DMA,
    ],
)
def _lookup(idx_hbm, packed_hbm, out_hbm, idx_v, pr_v, g_x, g_y, t_x, t_y,
            gsem_x, gsem_y, ssem_x, ssem_y):
    wid = lax.axis_index("s") * NC + lax.axis_index("c")

    pltpu.sync_copy(idx_hbm.at[pl.ds(wid * N_CH, N_CH)], idx_v)

    iota = lax.iota(jnp.int32, 16)

    def fire_gathers(sg, xy, buf, sem):
        hs = []
        for c in range(SUP):
            k = sg * SUP + c
            for jg in range(CH // 16):
                v = idx_v[k, pl.ds(jg * 16, 16)]
                pr_v[xy, c, pl.ds(jg * 16, 16)] = v // 4
            hs.append(pltpu.make_async_copy(
                packed_hbm.at[pr_v.at[xy, c]],
                buf.at[pl.ds(c * CH, CH)], sem))
        for h in hs:
            h.start()

    def drain_gathers(sg, xy, buf, sem):
        for c in range(SUP):
            pltpu.make_async_copy(
                packed_hbm.at[pr_v.at[xy, c]],
                buf.at[pl.ds(c * CH, CH)], sem).wait()

    def scatters(sg, tbuf, sem):
        hs = []
        for c in range(SUP):
            k = wid * N_CH + sg * SUP + c
            row_l = k // CH
            col = (k % CH) * CH
            hs.append(pltpu.make_async_copy(
                tbuf.at[c], out_hbm.at[row_l, :, pl.ds(col, CH)], sem))
        return hs

    def transpose(sg, gbuf, tbuf):
        for c in range(SUP):
            k = sg * SUP + c
            for jg in range(CH // 16):
                v = idx_v[k, pl.ds(jg * 16, 16)]
                colbase = (v % 4) * D
                rows = c * CH + jg * 16 + iota
                for d in range(D):
                    tbuf[c, d, pl.ds(jg * 16, 16)] = plsc.load_gather(
                        gbuf, [rows, colbase + d])

    fire_gathers(0, 0, g_x, gsem_x)

    def body(i, carry):
        s0 = 2 * i
        s1 = 2 * i + 1

        @pl.when(i > 0)
        def _():
            for h in scatters(s1 - 2, t_y, ssem_y):
                h.wait()

        fire_gathers(s1, 1, g_y, gsem_y)
        drain_gathers(s0, 0, g_x, gsem_x)

        @pl.when(i > 0)
        def _():
            for h in scatters(s0 - 2, t_x, ssem_x):
                h.wait()

        transpose(s0, g_x, t_x)
        for h in scatters(s0, t_x, ssem_x):
            h.start()

        @pl.when(i < N_PAIR - 1)
        def _():
            fire_gathers(s0 + 2, 0, g_x, gsem_x)

        drain_gathers(s1, 1, g_y, gsem_y)
        transpose(s1, g_y, t_y)
        for h in scatters(s1, t_y, ssem_y):
            h.start()
        return carry

    lax.fori_loop(0, N_PAIR, body, 0)

    for h in scatters(N_SUP - 2, t_x, ssem_x):
        h.wait()
    for h in scatters(N_SUP - 1, t_y, ssem_y):
        h.wait()


def kernel(vocab_ids, table):
    idx = jnp.transpose(vocab_ids).reshape(N // CH, CH)
    tail = jnp.pad(table[FULL_T * CH:], ((0, 0), (0, CH - D)))
    packed = _repack(jnp.transpose(table), tail.reshape(2, D, CH))
    out_t = _lookup(idx, packed)
    return out_t.transpose(2, 0, 1)


# speedup vs baseline: 1.1465x; 1.0755x over previous
import functools

import jax
import jax.numpy as jnp
from jax import lax
from jax.experimental import pallas as pl
from jax.experimental.pallas import tpu as pltpu
from jax.experimental.pallas import tpu_sc as plsc

B = 16384
L = 50
D = 32
N = B * L

_info = plsc.get_sparse_core_info()
NC, NS = _info.num_cores, _info.num_subcores
NW = NC * NS
PER_W = N // NW
CH = 128
N_CH = PER_W // CH
SUP = 5
ROWS_SUP = SUP * CH
N_SUP = N_CH // SUP
N_PAIR = N_SUP // 2

_mesh = plsc.VectorSubcoreMesh(core_axis_name="c", subcore_axis_name="s")


@functools.partial(
    pl.kernel,
    mesh=_mesh,
    compiler_params=pltpu.CompilerParams(
        use_tc_tiling_on_sc=False, needs_layout_passes=False),
    out_type=jax.ShapeDtypeStruct((L, D, B), jnp.float32),
    scratch_types=[
        pltpu.VMEM((N_CH, CH), jnp.int32),
        pltpu.VMEM((ROWS_SUP, D), jnp.float32),
        pltpu.VMEM((ROWS_SUP, D), jnp.float32),
        pltpu.VMEM((SUP, D, CH), jnp.float32),
        pltpu.VMEM((SUP, D, CH), jnp.float32),
        pltpu.SemaphoreType.DMA,
        pltpu.SemaphoreType.DMA,
        pltpu.SemaphoreType.DMA,
        pltpu.SemaphoreType.DMA,
    ],
)
def _sc_gather(idx_hbm, table_hbm, out_hbm, idx_v, g_x, g_y, t_x, t_y,
               gsem_x, gsem_y, ssem_x, ssem_y):
    wid = lax.axis_index("s") * NC + lax.axis_index("c")

    pltpu.sync_copy(idx_hbm.at[pl.ds(wid * N_CH, N_CH)], idx_v)

    iota = lax.iota(jnp.int32, 16)

    def gathers(sg, buf, sem):
        return [
            pltpu.make_async_copy(
                table_hbm.at[idx_v.at[sg * SUP + c]],
                buf.at[pl.ds(c * CH, CH)],
                sem,
            )
            for c in range(SUP)
        ]

    def scatters(sg, tbuf, sem):
        hs = []
        for c in range(SUP):
            k = wid * N_CH + sg * SUP + c
            row_l = k // CH
            col = (k % CH) * CH
            hs.append(pltpu.make_async_copy(
                tbuf.at[c], out_hbm.at[row_l, :, pl.ds(col, CH)], sem))
        return hs

    def transpose(gbuf, tbuf):
        def one_chunk(c, carry):
            for g in range(CH // 16):
                rows = c * CH + g * 16 + iota
                for d in range(D):
                    cols = jnp.full((16,), d, jnp.int32)
                    tbuf[c, d, pl.ds(g * 16, 16)] = plsc.load_gather(
                        gbuf, [rows, cols])
            return carry
        lax.fori_loop(0, SUP, one_chunk, 0)

    for h in gathers(0, g_x, gsem_x):
        h.start()

    def body(i, carry):
        s0 = 2 * i
        s1 = 2 * i + 1

        @pl.when(i > 0)
        def _():
            for h in scatters(s1 - 2, t_y, ssem_y):
                h.wait()

        for h in gathers(s1, g_y, gsem_y):
            h.start()

        for h in gathers(s0, g_x, gsem_x):
            h.wait()

        @pl.when(i > 0)
        def _():
            for h in scatters(s0 - 2, t_x, ssem_x):
                h.wait()

        transpose(g_x, t_x)
        for h in scatters(s0, t_x, ssem_x):
            h.start()

        @pl.when(i < N_PAIR - 1)
        def _():
            for h in gathers(s0 + 2, g_x, gsem_x):
                h.start()

        for h in gathers(s1, g_y, gsem_y):
            h.wait()
        transpose(g_y, t_y)
        for h in scatters(s1, t_y, ssem_y):
            h.start()
        return carry

    lax.fori_loop(0, N_PAIR, body, 0)

    for h in scatters(N_SUP - 2, t_x, ssem_x):
        h.wait()
    for h in scatters(N_SUP - 1, t_y, ssem_y):
        h.wait()


def kernel(vocab_ids, table):
    idx = jnp.transpose(vocab_ids).reshape(N // CH, CH)
    out_t = _sc_gather(idx, table)
    return out_t.transpose(2, 0, 1)
